# Initial kernel scaffold; baseline (speedup 1.0000x reference)
import jax
import jax.numpy as jnp
from jax.experimental import pallas as pl


def kernel(indices, offsets, tt_core_0, tt_core_1, tt_core_2):
    raise NotImplementedError("write your pallas kernel here")


# speedup vs baseline: 97.0462x; 97.0462x over previous
import functools

import jax
import jax.numpy as jnp
import numpy as np
from jax import lax
from jax.experimental import pallas as pl
from jax.experimental.pallas import tpu as pltpu
from jax.experimental.pallas import tpu_sc as plsc

P0, P1, P2 = 100, 100, 100
Q0, Q1, Q2 = 4, 4, 4
R1, R2 = 32, 32
EMB = 64
VOCAB = P0 * P1 * P2
NIDX = 327680
BATCH = 16384

NW = 32
PW = NIDX // NW
CH = 128
NCH = PW // CH
HALF = BATCH // 2
ACCROWS = HALF + 2048


_z = np.int32(0)


KT1 = R1 * Q1 * R2
NT1 = R1 * Q1 * Q2


def _t1_body(bm_ref, c2_ref, out_ref):
    c2 = c2_ref[0]
    rr = lax.broadcasted_iota(jnp.int32, (KT1, R2), 0)
    r2i = lax.broadcasted_iota(jnp.int32, (KT1, R2), 1)
    e1 = jnp.where(lax.rem(rr, jnp.full(rr.shape, R2, jnp.int32)) == r2i,
                   jnp.float32(1), jnp.float32(0))
    cc = lax.broadcasted_iota(jnp.int32, (Q2, NT1), 1)
    ci = lax.broadcasted_iota(jnp.int32, (Q2, NT1), 0)
    e2 = jnp.where(lax.rem(cc, jnp.full(cc.shape, Q2, jnp.int32)) == ci,
                   jnp.float32(1), jnp.float32(0))
    cbig = jnp.dot(jnp.dot(e1, c2, preferred_element_type=jnp.float32), e2,
                   preferred_element_type=jnp.float32)
    rk = lax.broadcasted_iota(jnp.int32, (KT1, NT1), 0)
    ck = lax.broadcasted_iota(jnp.int32, (KT1, NT1), 1)
    blk = lax.div(rk, jnp.full(rk.shape, R2, jnp.int32)) == \
        lax.div(ck, jnp.full(ck.shape, Q2, jnp.int32))
    w = jnp.where(blk, cbig, jnp.float32(0))
    out_ref[0] = jnp.dot(bm_ref[...], w, preferred_element_type=jnp.float32)


def _build_bct(bmat, c2r):
    return pl.pallas_call(
        _t1_body,
        grid=(P2,),
        in_specs=[
            pl.BlockSpec((P1, KT1), lambda i: (_z, _z)),
            pl.BlockSpec((1, R2, Q2), lambda i: (i, _z, _z)),
        ],
        out_specs=pl.BlockSpec((1, P1, NT1), lambda i: (i, _z, _z)),
        out_shape=jax.ShapeDtypeStruct((P2, P1, NT1), jnp.float32),
    )(bmat, c2r)


def _t2_body(bct_ref, a_ref, out_ref):
    at = a_ref[0]
    rr = lax.broadcasted_iota(jnp.int32, (NT1, R1), 0)
    r1i = lax.broadcasted_iota(jnp.int32, (NT1, R1), 1)
    e1 = jnp.where(lax.div(rr, jnp.full(rr.shape, 16, jnp.int32)) == r1i,
                   jnp.float32(1), jnp.float32(0))
    cc = lax.broadcasted_iota(jnp.int32, (Q0, EMB), 1)
    ai = lax.broadcasted_iota(jnp.int32, (Q0, EMB), 0)
    e2 = jnp.where(lax.div(cc, jnp.full(cc.shape, 16, jnp.int32)) == ai,
                   jnp.float32(1), jnp.float32(0))
    abig = jnp.dot(jnp.dot(e1, at, preferred_element_type=jnp.float32), e2,
                   preferred_element_type=jnp.float32)
    rk = lax.broadcasted_iota(jnp.int32, (NT1, EMB), 0)
    ck = lax.broadcasted_iota(jnp.int32, (NT1, EMB), 1)
    blk = lax.rem(rk, jnp.full(rk.shape, 16, jnp.int32)) == \
        lax.rem(ck, jnp.full(ck.shape, 16, jnp.int32))
    w = jnp.where(blk, abig, jnp.float32(0))
    out_ref[...] = jnp.dot(bct_ref[...], w, preferred_element_type=jnp.float32)


def _build_table(bct, a0r):
    return pl.pallas_call(
        _t2_body,
        grid=(P0,),
        in_specs=[
            pl.BlockSpec((P1 * P2, NT1), lambda i: (_z, _z)),
            pl.BlockSpec((1, R1, Q0), lambda i: (i, _z, _z)),
        ],
        out_specs=pl.BlockSpec((P1 * P2, EMB), lambda i: (i, _z)),
        out_shape=jax.ShapeDtypeStruct((VOCAB, EMB), jnp.float32),
    )(bct, a0r)


def _t3_body(p_ref, o_ref):
    o_ref[...] = p_ref[0] + p_ref[1]


def _sum_partials(partial):
    return pl.pallas_call(
        _t3_body,
        in_specs=[pl.BlockSpec((2, BATCH, EMB), lambda: (_z, _z, _z))],
        out_specs=pl.BlockSpec((BATCH, EMB), lambda: (_z, _z)),
        out_shape=jax.ShapeDtypeStruct((BATCH, EMB), jnp.float32),
    )(partial)


def _sc_bag_body(table_hbm, idx_hbm, offs_hbm, offsn_hbm, out_hbm,
                 offs_v, offsn_v, tmp_v, ridx_v, ridx2_v, idx_v, rows_v, sem, acc):
    cid = lax.axis_index("c")
    sid = lax.axis_index("s")
    wid = sid * 2 + cid
    base = wid * PW

    pltpu.sync_copy(offs_hbm, offs_v)
    pltpu.sync_copy(offsn_hbm, offsn_v)
    pltpu.sync_copy(idx_hbm.at[wid], idx_v)

    c8 = jnp.int32(8)
    c4 = jnp.int32(4)
    v10k = jnp.full((16,), P1 * P2, jnp.int32)
    v100 = jnp.full((16,), P2, jnp.int32)

    def xform(j, _):
        r_ = lax.div(j, c8)
        c_ = lax.rem(j, c8) * 16
        v = idx_v[r_, pl.ds(c_, 16)]
        i0 = lax.div(v, v10k)
        rm = lax.rem(v, v10k)
        i1 = lax.div(rm, v100)
        i2 = lax.rem(rm, v100)
        idx_v[r_, pl.ds(c_, 16)] = i0 * (P1 * P2) + i2 * P1 + i1
        return jnp.int32(0)
    lax.fori_loop(jnp.int32(0), jnp.int32(NCH * (CH // 16)), xform, jnp.int32(0))

    one_v = jnp.full((16,), 1, jnp.int32)
    zero_v = jnp.zeros((16,), jnp.int32)

    def cbody(k, acc_):
        ch = offs_v[pl.ds(k * 16, 16)]
        return acc_ + jnp.where(ch < base, one_v, zero_v)
    cnt = lax.fori_loop(jnp.int32(0), jnp.int32(BATCH // 16), cbody, jnp.zeros((16,), jnp.int32))
    carry0 = jnp.sum(cnt, dtype=jnp.int32) - 1

    def zbody(k, _):
        tmp_v[pl.ds(k * 16, 16)] = jnp.full((16,), -1, jnp.int32)
        return jnp.int32(0)
    lax.fori_loop(jnp.int32(0), jnp.int32(PW // 16), zbody, jnp.int32(0))

    lane = lax.iota(jnp.int32, 16)

    def sbody(k, _):
        o = offs_v[pl.ds(k * 16, 16)]
        onx = offsn_v[pl.ds(k * 16, 16)]
        bv = lane + k * 16
        rel = o - base
        m = (rel >= 0) & (rel < PW) & (o != onx)
        relc = jnp.clip(rel, 0, PW - 1)
        plsc.store_scatter(tmp_v, [relc], bv, mask=m)
        return jnp.int32(0)
    lax.fori_loop(jnp.int32(0), jnp.int32(BATCH // 16), sbody, jnp.int32(0))

    def mbody(k, cv):
        t = tmp_v[pl.ds(k * 16, 16)]
        m = jnp.maximum(plsc.cummax(t), cv)
        ridx_v[lax.div(k, c8), pl.ds(lax.rem(k, c8) * 16, 16)] = m
        return jnp.full((16,), jnp.max(m), jnp.int32)
    lax.fori_loop(jnp.int32(0), jnp.int32(PW // 16), mbody, jnp.full((16,), carry0, jnp.int32))

    zero16 = jnp.zeros((16,), jnp.float32)

    def zr(j, _):
        rows_v[lax.div(j, c4), pl.ds(lax.rem(j, c4) * 16, 16)] = zero16
        return jnp.int32(0)
    lax.fori_loop(jnp.int32(0), jnp.int32(CH * 4), zr, jnp.int32(0))

    vhalf = jnp.full((16,), HALF, jnp.int32)
    vdump = jnp.full((16,), 2048, jnp.int32)

    for p in range(2):
        def zs(j, _):
            pltpu.sync_copy(rows_v, acc.at[pl.ds(sid * (ACCROWS // 16) + j * CH, CH), :])
            return jnp.int32(0)
        lax.fori_loop(jnp.int32(0), jnp.int32(ACCROWS // 16 // CH), zs, jnp.int32(0))

        pbase = jnp.full((16,), p * HALF, jnp.int32)

        def rxf(j, _):
            r_ = lax.div(j, c8)
            c_ = lax.rem(j, c8) * 16
            rv = ridx_v[r_, pl.ds(c_, 16)]
            local = rv - pbase
            ok = (local >= 0) & (local < vhalf)
            spread = vhalf + lax.rem(rv, vdump)
            ridx2_v[r_, pl.ds(c_, 16)] = jnp.where(ok, local, spread)
            return jnp.int32(0)
        lax.fori_loop(jnp.int32(0), jnp.int32(NCH * (CH // 16)), rxf, jnp.int32(0))
        plsc.subcore_barrier()

        def gbody(c, _):
            pltpu.async_copy(table_hbm.at[idx_v.at[c]], rows_v, sem).wait()
            pltpu.sync_copy(rows_v, acc.at[ridx2_v.at[c]], add=True)
            return jnp.int32(0)
        lax.fori_loop(jnp.int32(0), jnp.int32(NCH), gbody, jnp.int32(0))
        plsc.subcore_barrier()

        def wb(j, _):
            row0 = sid * (HALF // 16) + j * CH
            pltpu.sync_copy(acc.at[pl.ds(row0, CH), :], rows_v)
            pltpu.sync_copy(rows_v, out_hbm.at[cid, pl.ds(p * HALF + row0, CH), :])
            return jnp.int32(0)
        lax.fori_loop(jnp.int32(0), jnp.int32(HALF // 16 // CH), wb, jnp.int32(0))
        plsc.subcore_barrier()

        lax.fori_loop(jnp.int32(0), jnp.int32(CH * 4), zr, jnp.int32(0))


def _sc_bag(table, idx3, offs_a, offs_b):
    mesh = plsc.VectorSubcoreMesh(core_axis_name="c", subcore_axis_name="s")
    fn = functools.partial(
        pl.kernel,
        mesh=mesh,
        compiler_params=pltpu.CompilerParams(
            needs_layout_passes=False, use_tc_tiling_on_sc=False),
        out_type=pltpu.HBM((2, BATCH, EMB), jnp.float32),
        scratch_types=[
            pltpu.VMEM((BATCH,), jnp.int32),
            pltpu.VMEM((BATCH,), jnp.int32),
            pltpu.VMEM((PW,), jnp.int32),
            pltpu.VMEM((NCH, CH), jnp.int32),
            pltpu.VMEM((NCH, CH), jnp.int32),
            pltpu.VMEM((NCH, CH), jnp.int32),
            pltpu.VMEM((CH, EMB), jnp.float32),
            pltpu.SemaphoreType.DMA,
            pltpu.VMEM_SHARED((ACCROWS, EMB), jnp.float32),
        ],
    )(_sc_bag_body)
    return fn(table, idx3, offs_a, offs_b)


def kernel(indices, offsets, tt_core_0, tt_core_1, tt_core_2):
    idx32 = indices.astype(jnp.int32)
    offs32 = offsets.astype(jnp.int32)

    bmat = tt_core_1
    c2r = tt_core_2.reshape(P2, R2, Q2)
    bct3 = _build_bct(bmat, c2r)
    bct = bct3.reshape(P1 * P2, NT1)

    a0r = jnp.transpose(tt_core_0.reshape(P0, Q0, R1), (0, 2, 1))
    table = _build_table(bct, a0r)

    idx3 = idx32.reshape(NW, NCH, CH)
    offs_a = offs32[:BATCH]
    offs_b = offs32[1:BATCH + 1]
    partial = _sc_bag(table, idx3, offs_a, offs_b)
    return _sum_partials(partial)
